# baseline (speedup 1.0000x reference)
import functools

import jax
import jax.numpy as jnp
from jax import lax
from jax.experimental import pallas as pl
from jax.experimental.pallas import tpu as pltpu
from jax.experimental.pallas import tpu_sc as plsc

B = 4096
D = 1024
DC = D // 16
NC = 2
NS = 16
NW = NC * NS
BPW = B // NW
CH = 16
NCH = BPW // CH


def _fold_body(w1a, b1a, w1b, b1b, w2a, b2a, w2b, b2b, wv):
    v1 = lax.dot_general(w1b[...], w1a[...], (((0,), (1,)), ((), ())))
    v2 = lax.dot_general(w2b[...], w2a[...], (((0,), (1,)), ((), ())))
    c1 = lax.dot_general(b1a[...], w1b[...], (((1,), (0,)), ((), ())))
    c2 = lax.dot_general(b2a[...], w2b[...], (((1,), (0,)), ((), ())))
    c = c1 + c2 + b1b[...] + b2b[...]
    wv[...] = jnp.concatenate(
        [v1, v1 + v2, jnp.broadcast_to(c, (1, D))], axis=0)


_fold_weights = pl.pallas_call(
    _fold_body,
    out_shape=jax.ShapeDtypeStruct((3, D), jnp.float32),
)


_mesh = plsc.VectorSubcoreMesh(core_axis_name="c", subcore_axis_name="s")


@functools.partial(
    pl.kernel,
    out_type=jax.ShapeDtypeStruct((B,), jnp.float32),
    mesh=_mesh,
    scratch_types=[
        pltpu.VMEM((BPW,), jnp.int32),
        pltpu.VMEM((3, D), jnp.float32),
        pltpu.VMEM((3, CH, D), jnp.float32),
        pltpu.VMEM((3, CH, D), jnp.float32),
        pltpu.VMEM((BPW,), jnp.float32),
        pltpu.SemaphoreType.DMA,
        pltpu.SemaphoreType.DMA,
        pltpu.SemaphoreType.DMA,
    ],
)
def _sc_gather_dot(x_hbm, t1_hbm, t2_hbm, wv_hbm, out_hbm,
                   idx_v, wv_v, t1_v, t2_v, out_v, sem1, sem2, sem3):
    wid = lax.axis_index("s") * NC + lax.axis_index("c")
    base = wid * BPW
    pltpu.sync_copy(x_hbm.at[pl.ds(base, BPW)], idx_v)
    pltpu.sync_copy(wv_hbm, wv_v)

    sems = (sem1, sem2, sem3)
    NBUF = 3

    def fire(ch, slot):
        s = sems[slot]
        c1 = pltpu.async_copy(
            t1_hbm.at[idx_v.at[pl.ds(ch * CH, CH)]], t1_v.at[slot], s)
        c2 = pltpu.async_copy(
            t2_hbm.at[idx_v.at[pl.ds(ch * CH, CH)]], t2_v.at[slot], s)
        return (c1, c2)

    zero = jnp.zeros((16,), jnp.float32)
    lane = lax.iota(jnp.int32, 16)
    perms = [lane ^ 8, lane ^ 4, lane ^ 2, lane ^ 1]
    c_v = wv_v[2, pl.ds(0, 16)]

    gdn = lax.GatherDimensionNumbers(
        offset_dims=(), collapsed_slice_dims=(0,), start_index_map=(0,))

    def _permute(a, p):
        return lax.gather(
            a, p[:, None], gdn, (1,),
            mode=lax.GatherScatterMode.PROMISE_IN_BOUNDS)

    def tree_total(a):
        for p in perms:
            a = a + _permute(a, p)
        return a

    pend = [None] * NBUF
    for ahead in range(NBUF - 1):
        pend[ahead] = fire(ahead, ahead)
    for ch in range(NCH):
        slot = ch % NBUF
        nxt = ch + NBUF - 1
        if nxt < NCH:
            pend[nxt % NBUF] = fire(nxt, nxt % NBUF)
        c1, c2 = pend[slot]
        c1.wait()
        c2.wait()

        def rb_body(rb, outvec, slot=slot):
            r0 = rb * 4

            def d_body(dd, accs):
                accs = list(accs)
                for u in range(2):
                    off = pl.multiple_of((dd * 2 + u) * 16, 16)
                    w1 = wv_v[0, pl.ds(off, 16)]
                    w2 = wv_v[1, pl.ds(off, 16)]
                    for j in range(4):
                        accs[j] = (accs[j]
                                   + t1_v[slot, r0 + j, pl.ds(off, 16)] * w1
                                   + t2_v[slot, r0 + j, pl.ds(off, 16)] * w2)
                return tuple(accs)

            accs = lax.fori_loop(0, DC // 2, d_body,
                                 (zero, zero, zero, zero))
            for j in range(4):
                outvec = jnp.where(lane == r0 + j, tree_total(accs[j]),
                                   outvec)
            return outvec

        outvec = lax.fori_loop(0, CH // 4, rb_body, zero)
        out_v[pl.ds(ch * CH, CH)] = outvec + c_v

    pltpu.sync_copy(out_v, out_hbm.at[pl.ds(base, BPW)])


def kernel(x, table_1, table_2, W1a, b1a, W1b, b1b, W2a, b2a, W2b, b2b):
    wv = _fold_weights(W1a, b1a.reshape(1, 512), W1b, b1b.reshape(1, 1),
                       W2a, b2a.reshape(1, 512), W2b, b2b.reshape(1, 1))
    out = _sc_gather_dot(x, table_1, table_2, wv)
    return out.reshape(B, 1)


# speedup vs baseline: 1.0893x; 1.0893x over previous
import functools

import jax
import jax.numpy as jnp
from jax import lax
from jax.experimental import pallas as pl
from jax.experimental.pallas import tpu as pltpu
from jax.experimental.pallas import tpu_sc as plsc

B = 4096
D = 1024
DC = D // 16
NC = 2
NS = 16
NW = NC * NS
BPW = B // NW
CH = 16
NCH = BPW // CH


def _fold_body(w1a, b1a, w1b, b1b, w2a, b2a, w2b, b2b, wv):
    v1 = lax.dot_general(w1b[...], w1a[...], (((0,), (1,)), ((), ())))
    v2 = lax.dot_general(w2b[...], w2a[...], (((0,), (1,)), ((), ())))
    c1 = lax.dot_general(b1a[...], w1b[...], (((1,), (0,)), ((), ())))
    c2 = lax.dot_general(b2a[...], w2b[...], (((1,), (0,)), ((), ())))
    c = c1 + c2 + b1b[...] + b2b[...]
    wv[...] = jnp.concatenate(
        [v1, v1 + v2, jnp.broadcast_to(c, (1, D))], axis=0)


_fold_weights = pl.pallas_call(
    _fold_body,
    out_shape=jax.ShapeDtypeStruct((3, D), jnp.float32),
)


_mesh = plsc.VectorSubcoreMesh(core_axis_name="c", subcore_axis_name="s")


@functools.partial(
    pl.kernel,
    out_type=jax.ShapeDtypeStruct((B,), jnp.float32),
    mesh=_mesh,
    scratch_types=[
        pltpu.VMEM((BPW,), jnp.int32),
        pltpu.VMEM((3, D), jnp.float32),
        pltpu.VMEM((3, CH, D), jnp.float32),
        pltpu.VMEM((3, CH, D), jnp.float32),
        pltpu.VMEM((BPW,), jnp.float32),
        pltpu.SemaphoreType.DMA,
        pltpu.SemaphoreType.DMA,
        pltpu.SemaphoreType.DMA,
    ],
)
def _sc_gather_dot(x_hbm, t1_hbm, t2_hbm, wv_hbm, out_hbm,
                   idx_v, wv_v, t1_v, t2_v, out_v, sem1, sem2, sem3):
    wid = lax.axis_index("s") * NC + lax.axis_index("c")
    base = wid * BPW
    pltpu.sync_copy(x_hbm.at[pl.ds(base, BPW)], idx_v)

    sems = (sem1, sem2, sem3)
    NBUF = 3

    def fire(ch, slot):
        s = sems[slot]
        c1 = pltpu.async_copy(
            t1_hbm.at[idx_v.at[pl.ds(ch * CH, CH)]], t1_v.at[slot], s)
        c2 = pltpu.async_copy(
            t2_hbm.at[idx_v.at[pl.ds(ch * CH, CH)]], t2_v.at[slot], s)
        return (c1, c2)

    pend = [None] * NBUF
    for ahead in range(NBUF - 1):
        pend[ahead] = fire(ahead, ahead)

    pltpu.sync_copy(wv_hbm, wv_v)

    zero = jnp.zeros((16,), jnp.float32)
    lane = lax.iota(jnp.int32, 16)
    perms = [lane ^ 8, lane ^ 4, lane ^ 2, lane ^ 1]
    c_v = wv_v[2, pl.ds(0, 16)]

    gdn = lax.GatherDimensionNumbers(
        offset_dims=(), collapsed_slice_dims=(0,), start_index_map=(0,))

    def _permute(a, p):
        return lax.gather(
            a, p[:, None], gdn, (1,),
            mode=lax.GatherScatterMode.PROMISE_IN_BOUNDS)

    def tree_total(a):
        for p in perms:
            a = a + _permute(a, p)
        return a

    for ch in range(NCH):
        slot = ch % NBUF
        nxt = ch + NBUF - 1
        if nxt < NCH:
            pend[nxt % NBUF] = fire(nxt, nxt % NBUF)
        c1, c2 = pend[slot]
        c1.wait()
        c2.wait()

        def d_body(d, accs, slot=slot):
            accs = list(accs)
            off = pl.multiple_of(d * 16, 16)
            w1 = wv_v[0, pl.ds(off, 16)]
            w2 = wv_v[1, pl.ds(off, 16)]
            for j in range(CH):
                accs[j] = (accs[j]
                           + t1_v[slot, j, pl.ds(off, 16)] * w1
                           + t2_v[slot, j, pl.ds(off, 16)] * w2)
            return tuple(accs)

        accs = lax.fori_loop(0, DC, d_body, (zero,) * CH)
        outvec = c_v
        for j in range(CH):
            outvec = jnp.where(lane == j, outvec + tree_total(accs[j]),
                               outvec)
        out_v[pl.ds(ch * CH, CH)] = outvec

    pltpu.sync_copy(out_v, out_hbm.at[pl.ds(base, BPW)])


def kernel(x, table_1, table_2, W1a, b1a, W1b, b1b, W2a, b2a, W2b, b2b):
    wv = _fold_weights(W1a, b1a.reshape(1, 512), W1b, b1b.reshape(1, 1),
                       W2a, b2a.reshape(1, 512), W2b, b2b.reshape(1, 1))
    out = _sc_gather_dot(x, table_1, table_2, wv)
    return out.reshape(B, 1)
